# baseline (speedup 1.0000x reference)
import functools

import jax
import jax.numpy as jnp
from jax import lax
from jax.experimental import pallas as pl
from jax.experimental.pallas import tpu as pltpu
from jax.experimental.pallas import tpu_sc as plsc

G, N = 4, 128
L, H, NH = 4, 32, 16
NB = 6
EDIM, NTE = 4, 2
V_SP, V_ACT, V_EDG, V_NT = 512, 128, 64, 128
P = G * N * N
D = L * H

TP = 1024
NTILE = P // TP
IB = TP // N
NIB = N // IB

NW = 32
CPW = P // NW
CH = 32
NCH = CPW // CH
NSRC = 8

NV = 4 * NB + 5
_C_BLNG = 0
_C_BLNB = NB
_C_B1 = 2 * NB
_C_B2 = 3 * NB
_C_NORMG = 4 * NB
_C_NORMB = 4 * NB + 1
_C_FC1B = 4 * NB + 2
_C_RESW = 4 * NB + 3
_C_RESB = 4 * NB + 4


def _gelu(x):
    return 0.5 * x * (1.0 + lax.erf(x * 0.7071067811865476))


def _ln_t(x, gcol, bcol):
    x3 = x.reshape(L, H, TP)
    mu = jnp.mean(x3, axis=1, keepdims=True)
    xc = x3 - mu
    var = jnp.mean(xc * xc, axis=1, keepdims=True)
    xn = (xc * lax.rsqrt(var + 1e-5)).reshape(L * H, TP)
    return xn * gcol + bcol


def _sc_body(comb_hbm, tab_hbm, out_hbm, tab_sh, idx_v, buf_v, sem0, sem1):
    sid = lax.axis_index("s")
    w = sid * 2 + lax.axis_index("c")

    @pl.when(sid == 0)
    def _():
        pltpu.sync_copy(tab_hbm, tab_sh)

    plsc.subcore_barrier()

    def fire(slot, c, sem):
        pltpu.sync_copy(comb_hbm.at[w, c], idx_v.at[slot])
        for k in range(NSRC):
            pltpu.async_copy(tab_sh.at[idx_v.at[slot, k]],
                             buf_v.at[slot, k], sem)

    def drain(slot, sem):
        for k in range(NSRC):
            pltpu.make_async_copy(tab_hbm.at[pl.ds(0, CH)],
                                  buf_v.at[slot, k], sem).wait()

    def process(slot, c):
        def row(r, carry2):
            for k in range(1, NSRC):
                for sub in range(D // 16):
                    sl = pl.ds(sub * 16, 16)
                    plsc.addupdate(buf_v.at[slot, 0, r, sl],
                                   buf_v[slot, k, r, sl])
            return carry2

        lax.fori_loop(0, CH, row, 0, unroll=False)
        pltpu.sync_copy(buf_v.at[slot, 0],
                        out_hbm.at[pl.ds(w * CPW + c * CH, CH)])

    fire(0, 0, sem0)

    def pair(cc, carry):
        c0 = cc * 2
        fire(1, c0 + 1, sem1)
        drain(0, sem0)
        process(0, c0)

        @pl.when(c0 + 2 < NCH)
        def _():
            fire(0, c0 + 2, sem0)

        drain(1, sem1)
        process(1, c0 + 1)
        return carry

    lax.fori_loop(0, NCH // 2, pair, 0, unroll=False)


def _sc_gather(comb4, tab):
    k = functools.partial(
        pl.kernel,
        out_type=jax.ShapeDtypeStruct((P, D), jnp.float32),
        mesh=plsc.VectorSubcoreMesh(core_axis_name="c", subcore_axis_name="s"),
        scratch_types=[
            pltpu.VMEM_SHARED((V_SP + V_ACT + V_EDG + V_NT, D), jnp.float32),
            pltpu.VMEM((2, NSRC, CH), jnp.int32),
            pltpu.VMEM((2, NSRC, CH, D), jnp.float32),
            pltpu.SemaphoreType.DMA,
            pltpu.SemaphoreType.DMA,
        ],
    )(_sc_body)
    return k(comb4, tab)


def _tc_body(acc_ref, pack_ref, mats_ref, vecs_ref, fc2t_ref, fc2b_ref, z_ref):
    f32 = jnp.float32
    rows = pack_ref[0]
    sp_row = rows[0:1]
    res_row = lax.bitcast_convert_type(rows[1:2], f32)

    xT = jnp.transpose(acc_ref[...])
    rp = vecs_ref[:, _C_RESW:_C_RESW + 1] * res_row \
        + vecs_ref[:, _C_RESB:_C_RESB + 1]
    x = (xT + rp) * 0.2

    for i in range(NB):
        h = _ln_t(x, vecs_ref[:, _C_BLNG + i:_C_BLNG + i + 1],
                  vecs_ref[:, _C_BLNB + i:_C_BLNB + i + 1])
        h = jnp.dot(mats_ref[i], h.astype(jnp.bfloat16),
                    preferred_element_type=f32) \
            + vecs_ref[:, _C_B1 + i:_C_B1 + i + 1]
        h = _gelu(h)
        h = jnp.dot(mats_ref[NB + i], h.astype(jnp.bfloat16),
                    preferred_element_type=f32) \
            + vecs_ref[:, _C_B2 + i:_C_B2 + i + 1]
        x = x + h

    x = _ln_t(x, vecs_ref[:, _C_NORMG:_C_NORMG + 1],
              vecs_ref[:, _C_NORMB:_C_NORMB + 1])
    x = _gelu(x)
    x = jnp.dot(mats_ref[2 * NB], x.astype(jnp.bfloat16),
                preferred_element_type=f32) \
        + vecs_ref[:, _C_FC1B:_C_FC1B + 1]
    x = _gelu(x)
    y = jnp.dot(fc2t_ref[...], x.astype(jnp.bfloat16),
                preferred_element_type=f32) + fc2b_ref[...]

    y = jnp.where(sp_row > 0, y, 0.0)
    z_ref[...] = y.reshape(L, NH, IB, N)[:, None]


@jax.jit
def kernel(spatial_pos, edge_long, action_pos, res_pos, node_type_edge,
           spatial_tab, action_tab, edge_tab, ntype_tab, res_w, res_b,
           bln_g, bln_b, bfc1_w, bfc1_b, bfc2_w, bfc2_b,
           norm_g, norm_b, fc1_w, fc1_b, fc2_w, fc2_b, t):
    f32 = jnp.float32
    flat = lambda a: a.reshape(-1)

    comb = jnp.stack([
        flat(spatial_pos),
        flat(action_pos) + V_SP,
        flat(edge_long[..., 0]) + (V_SP + V_ACT),
        flat(edge_long[..., 1]) + (V_SP + V_ACT),
        flat(edge_long[..., 2]) + (V_SP + V_ACT),
        flat(edge_long[..., 3]) + (V_SP + V_ACT),
        flat(node_type_edge[..., 0]) + (V_SP + V_ACT + V_EDG),
        flat(node_type_edge[..., 1]) + (V_SP + V_ACT + V_EDG),
    ])
    comb4 = comb.reshape(NSRC, NW, NCH, CH).transpose(1, 2, 0, 3)

    tab = jnp.concatenate([
        spatial_tab.at[0].set(0.0),
        action_tab.at[0].set(0.0),
        edge_tab.at[0].set(0.0) * 0.25,
        ntype_tab.at[0].set(0.0) * 0.5,
    ], axis=0)

    acc = _sc_gather(comb4, tab)

    pack = jnp.stack([
        flat(spatial_pos),
        lax.bitcast_convert_type(flat(res_pos), jnp.int32),
    ]).reshape(2, NTILE, TP).transpose(1, 0, 2)

    eye = jnp.eye(L, dtype=f32)
    bd = jax.vmap(lambda w: jnp.kron(eye, w.T))
    mats = jnp.concatenate([bd(bfc1_w), bd(bfc2_w),
                            jnp.kron(eye, fc1_w.T)[None]],
                           axis=0).astype(jnp.bfloat16)
    fc2t = jnp.kron(eye, fc2_w.T).astype(jnp.bfloat16)
    fc2b = jnp.tile(fc2_b, L)[:, None]

    tile4 = lambda v: jnp.tile(v, L)
    vec_cols = ([tile4(bln_g[i]) for i in range(NB)]
                + [tile4(bln_b[i]) for i in range(NB)]
                + [tile4(bfc1_b[i]) for i in range(NB)]
                + [tile4(bfc2_b[i]) for i in range(NB)]
                + [tile4(norm_g), tile4(norm_b), tile4(fc1_b),
                   res_w.reshape(-1), res_b])
    vecs = jnp.stack(vec_cols, axis=1)

    z = pl.pallas_call(
        _tc_body,
        grid=(G, NIB),
        in_specs=[
            pl.BlockSpec((TP, D), lambda g, ib: (g * NIB + ib, 0)),
            pl.BlockSpec((1, 2, TP), lambda g, ib: (g * NIB + ib, 0, 0)),
            pl.BlockSpec((2 * NB + 1, D, D), lambda g, ib: (0, 0, 0)),
            pl.BlockSpec((D, NV), lambda g, ib: (0, 0)),
            pl.BlockSpec((L * NH, D), lambda g, ib: (0, 0)),
            pl.BlockSpec((L * NH, 1), lambda g, ib: (0, 0)),
        ],
        out_specs=pl.BlockSpec((L, 1, NH, IB, N),
                               lambda g, ib: (0, g, 0, ib, 0)),
        out_shape=jax.ShapeDtypeStruct((L, G, NH, N, N), f32),
    )(acc, pack, mats, vecs, fc2t, fc2b)

    out = jnp.zeros((L, G, NH, N + 1, N + 1), dtype=f32)
    out = out.at[:, :, :, 1:, 1:].set(z)
    out = out.at[:, :, :, 0, 0].set(jnp.broadcast_to(t[0][:, None, :], (L, G, NH)))
    out = out.at[:, :, :, 0, 1:].set(
        jnp.broadcast_to(t[1][:, None, :, None], (L, G, NH, N)))
    out = out.at[:, :, :, 1:, 0].set(
        jnp.broadcast_to(t[2][:, None, :, None], (L, G, NH, N)))
    return out


# speedup vs baseline: 1.3742x; 1.3742x over previous
import functools

import jax
import jax.numpy as jnp
from jax import lax
from jax.experimental import pallas as pl
from jax.experimental.pallas import tpu as pltpu

G, N = 4, 128
L, H, NH = 4, 32, 16
NB = 6
EDIM, NTE = 4, 2
V_SP, V_ACT, V_EDG, V_NT = 512, 128, 64, 128

TP = 1024
NHALF = 4
TT = TP * NHALF
NTILE = G * N * N // TT
IB = TP // N
NIB = N // (IB * NHALF)

NV = 4 * NB + 5
_C_BLNG = 0
_C_BLNB = NB
_C_B1 = 2 * NB
_C_B2 = 3 * NB
_C_NORMG = 4 * NB
_C_NORMB = 4 * NB + 1
_C_FC1B = 4 * NB + 2
_C_RESW = 4 * NB + 3
_C_RESB = 4 * NB + 4


def _gelu(x):
    return 0.5 * x * (1.0 + lax.erf(x * 0.7071067811865476))


def _ln_t(x, gcol=None, bcol=None):
    x3 = x.reshape(L, H, TP)
    mu = jnp.mean(x3, axis=1, keepdims=True)
    xc = x3 - mu
    var = jnp.mean(xc * xc, axis=1, keepdims=True)
    xn = (xc * lax.rsqrt(var + 1e-5)).reshape(L * H, TP)
    if gcol is None:
        return xn
    return xn * gcol + bcol


def _half(rows, tsp, tac, ted, tnt, mats, vecs, fc2t, fc2b):
    f32 = jnp.float32
    sp_row = rows[0:1]

    def onehot(row, v):
        io = lax.broadcasted_iota(jnp.int32, (v, TP), 0)
        return (io == row).astype(f32)

    acc = jnp.dot(tsp, onehot(sp_row, V_SP), preferred_element_type=f32)
    acc += jnp.dot(tac, onehot(rows[1:2], V_ACT), preferred_element_type=f32)
    ed_cnt = (onehot(rows[2:3], V_EDG) + onehot(rows[3:4], V_EDG)
              + onehot(rows[4:5], V_EDG) + onehot(rows[5:6], V_EDG))
    acc += 0.25 * jnp.dot(ted, ed_cnt, preferred_element_type=f32)
    nt_cnt = onehot(rows[6:7], V_NT) + onehot(rows[7:8], V_NT)
    acc += 0.5 * jnp.dot(tnt, nt_cnt, preferred_element_type=f32)
    res_row = lax.bitcast_convert_type(rows[8:9], f32)
    acc += vecs[:, _C_RESW:_C_RESW + 1] * res_row \
        + vecs[:, _C_RESB:_C_RESB + 1]
    x = acc * 0.2

    for i in range(NB):
        h = _ln_t(x)
        h = jnp.dot(mats[i], h, preferred_element_type=f32) \
            + vecs[:, _C_B1 + i:_C_B1 + i + 1]
        h = _gelu(h)
        h = jnp.dot(mats[NB + i], h, preferred_element_type=f32) \
            + vecs[:, _C_B2 + i:_C_B2 + i + 1]
        x = x + h

    x = _ln_t(x, vecs[:, _C_NORMG:_C_NORMG + 1],
              vecs[:, _C_NORMB:_C_NORMB + 1])
    x = _gelu(x)
    x = jnp.dot(mats[2 * NB], x, preferred_element_type=f32) \
        + vecs[:, _C_FC1B:_C_FC1B + 1]
    x = _gelu(x)
    y = jnp.dot(fc2t, x, preferred_element_type=f32) + fc2b
    y = jnp.where(sp_row > 0, y, 0.0)
    return y.reshape(L, NH, IB, N)


def _body(idx_ref, tsp_ref, tac_ref, ted_ref, tnt_ref, mats_ref, vecs_ref,
          fc2t_ref, fc2b_ref, o_ref):
    args = (tsp_ref[...], tac_ref[...], ted_ref[...], tnt_ref[...],
            mats_ref, vecs_ref[...], fc2t_ref[...], fc2b_ref[...])
    ys = [_half(idx_ref[0, :, h * TP:(h + 1) * TP], *args)
          for h in range(NHALF)]
    o_ref[...] = jnp.concatenate(ys, axis=2)[:, None]


@jax.jit
def kernel(spatial_pos, edge_long, action_pos, res_pos, node_type_edge,
           spatial_tab, action_tab, edge_tab, ntype_tab, res_w, res_b,
           bln_g, bln_b, bfc1_w, bfc1_b, bfc2_w, bfc2_b,
           norm_g, norm_b, fc1_w, fc1_b, fc2_w, fc2_b, t):
    f32 = jnp.float32

    flat = lambda a: a.reshape(-1)
    idx_rows = jnp.stack([
        flat(spatial_pos), flat(action_pos),
        flat(edge_long[..., 0]), flat(edge_long[..., 1]),
        flat(edge_long[..., 2]), flat(edge_long[..., 3]),
        flat(node_type_edge[..., 0]), flat(node_type_edge[..., 1]),
        lax.bitcast_convert_type(flat(res_pos), jnp.int32),
    ])
    idx_pack = idx_rows.reshape(9, NTILE, TT).transpose(1, 0, 2)

    tsp = spatial_tab.at[0].set(0.0).T
    tac = action_tab.at[0].set(0.0).T
    ted = edge_tab.at[0].set(0.0).T
    tnt = ntype_tab.at[0].set(0.0).T

    eye = jnp.eye(L, dtype=f32)
    bd = jax.vmap(lambda w: jnp.kron(eye, w.T))
    w1g = bln_g[:, :, None] * bfc1_w
    mats = jnp.concatenate([bd(w1g), bd(bfc2_w),
                            jnp.kron(eye, fc1_w.T)[None]],
                           axis=0)
    fc2t = jnp.kron(eye, fc2_w.T)
    fc2b = jnp.tile(fc2_b, L)[:, None]

    tile4 = lambda v: jnp.tile(v, L)
    vec_cols = ([tile4(bln_g[i]) for i in range(NB)]
                + [tile4(bln_b[i]) for i in range(NB)]
                + [tile4(bfc1_b[i] + bln_b[i] @ bfc1_w[i]) for i in range(NB)]
                + [tile4(bfc2_b[i]) for i in range(NB)]
                + [tile4(norm_g), tile4(norm_b), tile4(fc1_b),
                   res_w.reshape(-1), res_b])
    vecs = jnp.stack(vec_cols, axis=1)

    grid = (G, NIB)
    z = pl.pallas_call(
        _body,
        grid=grid,
        in_specs=[
            pl.BlockSpec((1, 9, TT), lambda g, ib: (g * NIB + ib, 0, 0)),
            pl.BlockSpec((L * H, V_SP), lambda g, ib: (0, 0)),
            pl.BlockSpec((L * H, V_ACT), lambda g, ib: (0, 0)),
            pl.BlockSpec((L * H, V_EDG), lambda g, ib: (0, 0)),
            pl.BlockSpec((L * H, V_NT), lambda g, ib: (0, 0)),
            pl.BlockSpec((2 * NB + 1, L * H, L * H), lambda g, ib: (0, 0, 0)),
            pl.BlockSpec((L * H, NV), lambda g, ib: (0, 0)),
            pl.BlockSpec((L * NH, L * H), lambda g, ib: (0, 0)),
            pl.BlockSpec((L * NH, 1), lambda g, ib: (0, 0)),
        ],
        out_specs=pl.BlockSpec((L, 1, NH, IB * NHALF, N),
                               lambda g, ib: (0, g, 0, ib, 0)),
        out_shape=jax.ShapeDtypeStruct((L, G, NH, N, N), f32),
    )(idx_pack, tsp, tac, ted, tnt, mats, vecs, fc2t, fc2b)

    out = jnp.zeros((L, G, NH, N + 1, N + 1), dtype=f32)
    out = out.at[:, :, :, 1:, 1:].set(z)
    out = out.at[:, :, :, 0, 0].set(jnp.broadcast_to(t[0][:, None, :], (L, G, NH)))
    out = out.at[:, :, :, 0, 1:].set(
        jnp.broadcast_to(t[1][:, None, :, None], (L, G, NH, N)))
    out = out.at[:, :, :, 1:, 0].set(
        jnp.broadcast_to(t[2][:, None, :, None], (L, G, NH, N)))
    return out
